# Initial kernel scaffold; baseline (speedup 1.0000x reference)
import jax
import jax.numpy as jnp
from jax.experimental import pallas as pl


def kernel(x, adj, W1, b1, W_mu, b_mu, W_ls, b_ls):
    raise NotImplementedError("write your pallas kernel here")


# speedup vs baseline: 1.0930x; 1.0930x over previous
import functools

import jax
import jax.numpy as jnp
from jax.experimental import pallas as pl
from jax.experimental.pallas import tpu as pltpu


def _linear_kernel(x_ref, wt_ref, b_ref, o_ref):
    o_ref[:] = (
        jnp.dot(x_ref[:], wt_ref[:], preferred_element_type=jnp.float32)
        + b_ref[:]
    )


def _spmm_tanh_kernel(adj_ref, u_ref, o_ref):
    o_ref[:] = jnp.tanh(
        jnp.dot(adj_ref[:], u_ref[:], preferred_element_type=jnp.float32))


def _spmm_head_kernel(adj_ref, g_ref, eps_ref, mu_ref, ls_ref, z_ref, *, o_dim):
    ml = jnp.dot(adj_ref[:], g_ref[:], preferred_element_type=jnp.float32)
    mu = ml[:, :o_dim]
    ls = ml[:, o_dim:]
    mu_ref[:] = mu
    ls_ref[:] = ls
    z_ref[:] = mu + eps_ref[:] * jnp.exp(ls)


def _gram_kernel(zi_ref, zj_ref, o_ref):
    o_ref[:] = jax.lax.dot_general(
        zi_ref[:], zj_ref[:], (((1,), (1,)), ((), ())),
        preferred_element_type=jnp.float32,
    )


def kernel(x, adj, W1, b1, W_mu, b_mu, W_ls, b_ls):
    n, f_in = x.shape
    h_dim = W1.shape[0]
    o_dim = W_mu.shape[0]

    eps = jax.random.normal(jax.random.key(42), (n, o_dim), dtype=jnp.float32)

    w1_t = W1.T
    b1_r = b1.reshape(1, h_dim)
    wcat_t = jnp.concatenate([W_mu, W_ls], axis=0).T
    bcat_r = jnp.concatenate([b_mu, b_ls]).reshape(1, 2 * o_dim)

    BM = 400
    nm = n // BM

    def linear(inp, wt, b, out_dim):
        return pl.pallas_call(
            _linear_kernel,
            grid=(nm,),
            in_specs=[
                pl.BlockSpec((BM, inp.shape[1]), lambda m: (m, 0)),
                pl.BlockSpec(wt.shape, lambda m: (0, 0)),
                pl.BlockSpec((1, out_dim), lambda m: (0, 0)),
            ],
            out_specs=pl.BlockSpec((BM, out_dim), lambda m: (m, 0)),
            out_shape=jax.ShapeDtypeStruct((n, out_dim), jnp.float32),
        )(inp, wt, b)

    u = linear(x, w1_t, b1_r, h_dim)

    h = pl.pallas_call(
        _spmm_tanh_kernel,
        grid=(nm,),
        in_specs=[
            pl.BlockSpec((BM, n), lambda m: (m, 0)),
            pl.BlockSpec((n, h_dim), lambda m: (0, 0)),
        ],
        out_specs=pl.BlockSpec((BM, h_dim), lambda m: (m, 0)),
        out_shape=jax.ShapeDtypeStruct((n, h_dim), jnp.float32),
        compiler_params=pltpu.CompilerParams(
            dimension_semantics=("arbitrary",)),
    )(adj, u)

    g = linear(h, wcat_t, bcat_r, 2 * o_dim)

    mu, logstd, z = pl.pallas_call(
        functools.partial(_spmm_head_kernel, o_dim=o_dim),
        grid=(nm,),
        in_specs=[
            pl.BlockSpec((BM, n), lambda m: (m, 0)),
            pl.BlockSpec((n, 2 * o_dim), lambda m: (0, 0)),
            pl.BlockSpec((BM, o_dim), lambda m: (m, 0)),
        ],
        out_specs=[
            pl.BlockSpec((BM, o_dim), lambda m: (m, 0)),
            pl.BlockSpec((BM, o_dim), lambda m: (m, 0)),
            pl.BlockSpec((BM, o_dim), lambda m: (m, 0)),
        ],
        out_shape=[
            jax.ShapeDtypeStruct((n, o_dim), jnp.float32),
            jax.ShapeDtypeStruct((n, o_dim), jnp.float32),
            jax.ShapeDtypeStruct((n, o_dim), jnp.float32),
        ],
        compiler_params=pltpu.CompilerParams(
            dimension_semantics=("arbitrary",)),
    )(adj, g, eps)

    adj_logits = pl.pallas_call(
        _gram_kernel,
        grid=(nm,),
        in_specs=[
            pl.BlockSpec((BM, o_dim), lambda m: (m, 0)),
            pl.BlockSpec((n, o_dim), lambda m: (0, 0)),
        ],
        out_specs=pl.BlockSpec((BM, n), lambda m: (m, 0)),
        out_shape=jax.ShapeDtypeStruct((n, n), jnp.float32),
        compiler_params=pltpu.CompilerParams(
            dimension_semantics=("arbitrary",)),
    )(z, z)

    return (z, adj_logits, mu, logstd)
